# baseline (speedup 1.0000x reference)
import functools

import jax
import jax.numpy as jnp
from jax import lax
from jax.experimental import pallas as pl
from jax.experimental.pallas import tpu as pltpu
from jax.experimental.pallas import tpu_sc as plsc

E = 8
TOPK = 2
DMODEL = 1024
DFF = 512
NGROUPS = 2
GROUP_SIZE = E // NGROUPS
N_TOKENS = 2048

_SC_INFO = plsc.get_sparse_core_info()
_NC = _SC_INFO.num_cores
_NS = _SC_INFO.num_subcores
_LANES = _SC_INFO.num_lanes

T_BLK = 256
PG = N_TOKENS + GROUP_SIZE * T_BLK
PTOT = NGROUPS * PG
NBG = PG // T_BLK
NB = NGROUPS * NBG
TOK_PER_TILE = N_TOKENS // _NS
ROWS_PER_TILE = PG // _NS
GCH = 64
N_GCH = ROWS_PER_TILE // GCH
TOK_D = N_TOKENS // (_NC * _NS)
CCH = 32


def _logits_kernel(x_ref, gw_ref, lt_ref):
    lt_ref[...] = jax.lax.dot_general(
        gw_ref[...], x_ref[...], (((0,), (1,)), ((), ())),
        preferred_element_type=jnp.float32)


def _sc_dispatch_kernel(lt_hbm, x_hbm, xs_hbm, pos_hbm, w_hbm, be_hbm,
                        lt_v, aid_v, wt_v, aid_all_v, stok_v, pos_v, be_v,
                        idx_v, rows_v, aid_sh, stok_sh, sem):
    g = lax.axis_index("c")
    s = lax.axis_index("s")
    base = s * TOK_PER_TILE
    own = g * GROUP_SIZE
    oth = (1 - g) * GROUP_SIZE
    for e in range(GROUP_SIZE):
        pltpu.sync_copy(lt_hbm.at[own + e, pl.ds(base, TOK_PER_TILE)],
                        lt_v.at[e])
        pltpu.sync_copy(lt_hbm.at[oth + e, pl.ds(base, TOK_PER_TILE)],
                        lt_v.at[GROUP_SIZE + e])
    for j in range(TOK_PER_TILE // _LANES):
        sl = pl.ds(j * _LANES, _LANES)
        l = [lt_v[e, sl] for e in range(E)]
        m0 = l[0]
        a0 = jnp.full((_LANES,), 0, jnp.int32)
        for i in range(1, GROUP_SIZE):
            gt = l[i] > m0
            a0 = jnp.where(gt, i, a0)
            m0 = jnp.where(gt, l[i], m0)
        m1 = l[GROUP_SIZE]
        for i in range(1, GROUP_SIZE):
            m1 = jnp.maximum(m1, l[GROUP_SIZE + i])
        mm = jnp.maximum(m0, m1)
        e0 = jnp.exp(m0 - mm)
        e1 = jnp.exp(m1 - mm)
        aid_v[sl] = a0
        wt_v[sl] = e0 / (e0 + e1)
    pltpu.sync_copy(aid_v, aid_sh.at[pl.ds(base, TOK_PER_TILE)])
    pltpu.sync_copy(wt_v, w_hbm.at[g, pl.ds(base, TOK_PER_TILE)])
    plsc.subcore_barrier()

    @pl.when(s == 0)
    def _():
        pltpu.sync_copy(aid_sh, aid_all_v)

        def _ms(i, c):
            stok_v[pl.ds(i * _LANES, _LANES)] = jnp.full(
                (_LANES,), 0, jnp.int32)
            return c
        lax.fori_loop(0, PG // _LANES, _ms, 0)

        def _cnt(j, carry):
            a = aid_all_v[pl.ds(j * _LANES, _LANES)]
            return tuple(
                carry[e] + jnp.sum((a == e).astype(jnp.int32))
                for e in range(GROUP_SIZE))
        zero = jnp.array(0, jnp.int32)
        cnts = lax.fori_loop(0, N_TOKENS // _LANES, _cnt,
                             (zero, zero, zero, zero))
        pads = [jnp.bitwise_and(c + (T_BLK - 1), -T_BLK) for c in cnts]
        starts = [zero, pads[0], pads[0] + pads[1], pads[0] + pads[1] + pads[2]]

        def _sct(j, bases):
            a = aid_all_v[pl.ds(j * _LANES, _LANES)]
            tok = j * _LANES + lax.iota(jnp.int32, _LANES)
            posf = jnp.full((_LANES,), 0, jnp.int32)
            nb = []
            for e in range(GROUP_SIZE):
                m = a == e
                m01 = m.astype(jnp.int32)
                pv = bases[e] + jnp.cumsum(m01) - 1
                plsc.store_scatter(stok_v, [pv], tok, mask=m)
                posf = jnp.where(m, pv, posf)
                nb.append(bases[e] + jnp.sum(m01))
            pos_v[pl.ds(j * _LANES, _LANES)] = posf + g * PG
            return tuple(nb)
        lax.fori_loop(0, N_TOKENS // _LANES, _sct, tuple(starts))

        row = lax.iota(jnp.int32, _LANES) * T_BLK
        eb = jnp.full((_LANES,), 0, jnp.int32)
        for e in range(1, GROUP_SIZE):
            eb += (row >= starts[e]).astype(jnp.int32)
        be_v[...] = eb + g * GROUP_SIZE
        pltpu.sync_copy(stok_v, stok_sh)
        pltpu.sync_copy(pos_v, pos_hbm.at[g])
        pltpu.sync_copy(be_v, be_hbm.at[g])
    plsc.subcore_barrier()

    rbase = s * ROWS_PER_TILE
    for k in range(N_GCH):
        pltpu.sync_copy(stok_sh.at[pl.ds(rbase + k * GCH, GCH)], idx_v.at[k])
    for k in range(N_GCH):
        pltpu.async_copy(x_hbm.at[idx_v.at[k]], rows_v, sem).wait()
        pltpu.sync_copy(rows_v,
                        xs_hbm.at[pl.ds(g * PG + rbase + k * GCH, GCH), :])


def _sc_dispatch(lt, x):
    mesh = plsc.VectorSubcoreMesh(core_axis_name="c", subcore_axis_name="s")
    f = functools.partial(
        pl.kernel,
        mesh=mesh,
        compiler_params=pltpu.CompilerParams(needs_layout_passes=False),
        out_type=(
            jax.ShapeDtypeStruct((PTOT, DMODEL), jnp.float32),
            jax.ShapeDtypeStruct((NGROUPS, N_TOKENS), jnp.int32),
            jax.ShapeDtypeStruct((NGROUPS, N_TOKENS), jnp.float32),
            jax.ShapeDtypeStruct((NGROUPS, _LANES), jnp.int32),
        ),
        scratch_types=[
            pltpu.VMEM((E, TOK_PER_TILE), jnp.float32),
            pltpu.VMEM((TOK_PER_TILE,), jnp.int32),
            pltpu.VMEM((TOK_PER_TILE,), jnp.float32),
            pltpu.VMEM((N_TOKENS,), jnp.int32),
            pltpu.VMEM((PG,), jnp.int32),
            pltpu.VMEM((N_TOKENS,), jnp.int32),
            pltpu.VMEM((_LANES,), jnp.int32),
            pltpu.VMEM((N_GCH, GCH), jnp.int32),
            pltpu.VMEM((GCH, DMODEL), jnp.float32),
            pltpu.VMEM_SHARED((N_TOKENS,), jnp.int32),
            pltpu.VMEM_SHARED((PG,), jnp.int32),
            pltpu.SemaphoreType.DMA,
        ],
    )(_sc_dispatch_kernel)
    return f(lt, x)


def _gffn_kernel(be_ref, xs_ref, w1_ref, w3_ref, w2_ref, ys_ref):
    xb = xs_ref[...].astype(jnp.bfloat16)
    a = jax.lax.dot_general(xb, w1_ref[0].astype(jnp.bfloat16),
                            (((1,), (1,)), ((), ())),
                            preferred_element_type=jnp.float32)
    b = jax.lax.dot_general(xb, w3_ref[0].astype(jnp.bfloat16),
                            (((1,), (1,)), ((), ())),
                            preferred_element_type=jnp.float32)
    h = (a * jax.nn.sigmoid(a) * b).astype(jnp.bfloat16)
    ys_ref[...] = jax.lax.dot_general(h, w2_ref[0].astype(jnp.bfloat16),
                                      (((1,), (1,)), ((), ())),
                                      preferred_element_type=jnp.float32)


def _sc_combine_kernel(ys_hbm, pos_hbm, w_hbm, out_hbm,
                       p_v, wv_v, r0_v, r1_v, o_v, sem):
    wid = lax.axis_index("s") * _NC + lax.axis_index("c")
    tb = wid * TOK_D
    for gg in range(NGROUPS):
        pltpu.sync_copy(pos_hbm.at[gg, pl.ds(tb, TOK_D)], p_v.at[gg])
        pltpu.sync_copy(w_hbm.at[gg, pl.ds(tb, TOK_D)], wv_v.at[gg])
    for ch in range(TOK_D // CCH):
        pltpu.async_copy(ys_hbm.at[p_v.at[0, pl.ds(ch * CCH, CCH)]],
                         r0_v, sem).wait()
        pltpu.async_copy(ys_hbm.at[p_v.at[1, pl.ds(ch * CCH, CCH)]],
                         r1_v, sem).wait()
        wvec0 = [wv_v[0, pl.ds(ch * CCH + q * _LANES, _LANES)]
                 for q in range(CCH // _LANES)]
        wvec1 = [wv_v[1, pl.ds(ch * CCH + q * _LANES, _LANES)]
                 for q in range(CCH // _LANES)]
        for r in range(CCH):
            w0s = wvec0[r // _LANES][r % _LANES]
            w1s = wvec1[r // _LANES][r % _LANES]

            def _row(cc, c, r=r, w0s=w0s, w1s=w1s):
                sl = pl.ds(cc * _LANES, _LANES)
                o_v[r, sl] = w0s * r0_v[r, sl] + w1s * r1_v[r, sl]
                return c
            lax.fori_loop(0, DMODEL // _LANES, _row, 0)
        pltpu.sync_copy(o_v, out_hbm.at[pl.ds(tb + ch * CCH, CCH), :])


def _sc_combine(ys, pos, w):
    mesh = plsc.VectorSubcoreMesh(core_axis_name="c", subcore_axis_name="s")
    f = functools.partial(
        pl.kernel,
        mesh=mesh,
        compiler_params=pltpu.CompilerParams(needs_layout_passes=False),
        out_type=jax.ShapeDtypeStruct((N_TOKENS, DMODEL), jnp.float32),
        scratch_types=[
            pltpu.VMEM((NGROUPS, TOK_D), jnp.int32),
            pltpu.VMEM((NGROUPS, TOK_D), jnp.float32),
            pltpu.VMEM((CCH, DMODEL), jnp.float32),
            pltpu.VMEM((CCH, DMODEL), jnp.float32),
            pltpu.VMEM((CCH, DMODEL), jnp.float32),
            pltpu.SemaphoreType.DMA,
        ],
    )(_sc_combine_kernel)
    return f(ys, pos, w)


@jax.jit
def kernel(hidden_states, gate_w, w1, w3, w2):
    lt = pl.pallas_call(
        _logits_kernel,
        out_shape=jax.ShapeDtypeStruct((E, N_TOKENS), jnp.float32),
    )(hidden_states, gate_w)

    xs, pos, w, be = _sc_dispatch(lt, hidden_states)

    ys = pl.pallas_call(
        _gffn_kernel,
        grid_spec=pltpu.PrefetchScalarGridSpec(
            num_scalar_prefetch=1,
            grid=(NB,),
            in_specs=[
                pl.BlockSpec((T_BLK, DMODEL), lambda i, be: (i, 0)),
                pl.BlockSpec((1, DFF, DMODEL),
                             lambda i, be: (be[i // NBG, i % NBG], 0, 0)),
                pl.BlockSpec((1, DFF, DMODEL),
                             lambda i, be: (be[i // NBG, i % NBG], 0, 0)),
                pl.BlockSpec((1, DMODEL, DFF),
                             lambda i, be: (be[i // NBG, i % NBG], 0, 0)),
            ],
            out_specs=pl.BlockSpec((T_BLK, DMODEL), lambda i, be: (i, 0)),
        ),
        out_shape=jax.ShapeDtypeStruct((PTOT, DMODEL), jnp.float32),
    )(be, xs, w1, w3, w2)

    return _sc_combine(ys, pos, w)


# speedup vs baseline: 1.0076x; 1.0076x over previous
import functools

import jax
import jax.numpy as jnp
from jax import lax
from jax.experimental import pallas as pl
from jax.experimental.pallas import tpu as pltpu
from jax.experimental.pallas import tpu_sc as plsc

E = 8
TOPK = 2
DMODEL = 1024
DFF = 512
NGROUPS = 2
GROUP_SIZE = E // NGROUPS
N_TOKENS = 2048

_SC_INFO = plsc.get_sparse_core_info()
_NC = _SC_INFO.num_cores
_NS = _SC_INFO.num_subcores
_LANES = _SC_INFO.num_lanes

T_BLK = 256
PG = N_TOKENS + GROUP_SIZE * T_BLK
PTOT = NGROUPS * PG
NBG = PG // T_BLK
NB = NGROUPS * NBG
TOK_PER_TILE = N_TOKENS // _NS
ROWS_PER_TILE = PG // _NS
GCH = 48
N_GCH = ROWS_PER_TILE // GCH
TOK_D = N_TOKENS // (_NC * _NS)
CCH = 32
_UNR = 8


def _logits_kernel(x_ref, gw_ref, lt_ref):
    lt = jax.lax.dot_general(
        gw_ref[...], x_ref[...], (((0,), (1,)), ((), ())),
        preferred_element_type=jnp.float32)
    for s in range(_NS):
        lt_ref[s] = lt[:, s * TOK_PER_TILE:(s + 1) * TOK_PER_TILE]


def _sc_dispatch_kernel(lt3_hbm, x_hbm, xs_hbm, pos_hbm, w_hbm, be_hbm,
                        lt_v, aid_v, wt_v, aid_all_v, stok_v, pos_v, be_v,
                        idx_v, rows_a, rows_b, aid_sh, stok_sh, sem, semg):
    g = lax.axis_index("c")
    s = lax.axis_index("s")
    base = s * TOK_PER_TILE
    pltpu.sync_copy(lt3_hbm.at[s], lt_v)
    gz = g == 0
    for j in range(TOK_PER_TILE // _LANES):
        sl = pl.ds(j * _LANES, _LANES)
        l = [lt_v[e, sl] for e in range(E)]
        m0 = l[0]
        a0 = jnp.full((_LANES,), 0, jnp.int32)
        m1 = l[GROUP_SIZE]
        a1 = jnp.full((_LANES,), 0, jnp.int32)
        for i in range(1, GROUP_SIZE):
            gt0 = l[i] > m0
            a0 = jnp.where(gt0, i, a0)
            m0 = jnp.where(gt0, l[i], m0)
            gt1 = l[GROUP_SIZE + i] > m1
            a1 = jnp.where(gt1, i, a1)
            m1 = jnp.where(gt1, l[GROUP_SIZE + i], m1)
        mm = jnp.maximum(m0, m1)
        e0 = jnp.exp(m0 - mm)
        e1 = jnp.exp(m1 - mm)
        den = e0 + e1
        aid_v[sl] = jnp.where(gz, a0, a1)
        wt_v[sl] = jnp.where(gz, e0 / den, e1 / den)
    pltpu.sync_copy(aid_v, aid_sh.at[pl.ds(base, TOK_PER_TILE)])
    pltpu.sync_copy(wt_v, w_hbm.at[g, pl.ds(base, TOK_PER_TILE)])
    plsc.subcore_barrier()

    @pl.when(s == 0)
    def _():
        pltpu.sync_copy(aid_sh, aid_all_v)

        def _ms(i, c):
            stok_v[pl.ds(i * _LANES, _LANES)] = jnp.full(
                (_LANES,), 0, jnp.int32)
            return c
        lax.fori_loop(0, PG // _LANES, _ms, 0)

        def _cnt(j, carry):
            a = aid_all_v[pl.ds(j * _LANES, _LANES)]
            return tuple(
                carry[e] + jnp.sum((a == e).astype(jnp.int32))
                for e in range(GROUP_SIZE))
        zero = jnp.array(0, jnp.int32)
        cnts = lax.fori_loop(0, N_TOKENS // _LANES, _cnt,
                             (zero,) * GROUP_SIZE)
        pads = [jnp.bitwise_and(c + (T_BLK - 1), -T_BLK) for c in cnts]
        starts = [zero, pads[0], pads[0] + pads[1],
                  pads[0] + pads[1] + pads[2]]

        def _sct(j, bases):
            a = aid_all_v[pl.ds(j * _LANES, _LANES)]
            tok = j * _LANES + lax.iota(jnp.int32, _LANES)
            posf = jnp.full((_LANES,), 0, jnp.int32)
            nb = []
            for e in range(GROUP_SIZE):
                m = a == e
                m01 = m.astype(jnp.int32)
                pv = bases[e] + jnp.cumsum(m01) - 1
                plsc.store_scatter(stok_v, [pv], tok, mask=m)
                posf = jnp.where(m, pv, posf)
                nb.append(bases[e] + jnp.sum(m01))
            pos_v[pl.ds(j * _LANES, _LANES)] = posf + g * PG
            return tuple(nb)
        lax.fori_loop(0, N_TOKENS // _LANES, _sct, tuple(starts))

        row = lax.iota(jnp.int32, _LANES) * T_BLK
        eb = jnp.full((_LANES,), 0, jnp.int32)
        for e in range(1, GROUP_SIZE):
            eb += (row >= starts[e]).astype(jnp.int32)
        be_v[...] = eb + g * GROUP_SIZE
        pltpu.sync_copy(stok_v, stok_sh)
        pltpu.sync_copy(pos_v, pos_hbm.at[g])
        pltpu.sync_copy(be_v, be_hbm.at[g])
    plsc.subcore_barrier()

    rbase = s * ROWS_PER_TILE
    for k in range(N_GCH):
        pltpu.sync_copy(stok_sh.at[pl.ds(rbase + k * GCH, GCH)], idx_v.at[k])
    for k in range(N_GCH):
        pltpu.async_copy(x_hbm.at[idx_v.at[k]], rows_a, semg).wait()
        pltpu.sync_copy(
            rows_a, xs_hbm.at[pl.ds(g * PG + rbase + k * GCH, GCH), :])


def _sc_dispatch(lt3, x):
    mesh = plsc.VectorSubcoreMesh(core_axis_name="c", subcore_axis_name="s")
    f = functools.partial(
        pl.kernel,
        mesh=mesh,
        compiler_params=pltpu.CompilerParams(needs_layout_passes=False),
        out_type=(
            jax.ShapeDtypeStruct((PTOT, DMODEL), jnp.float32),
            jax.ShapeDtypeStruct((NGROUPS, N_TOKENS), jnp.int32),
            jax.ShapeDtypeStruct((NGROUPS, N_TOKENS), jnp.float32),
            jax.ShapeDtypeStruct((NGROUPS, _LANES), jnp.int32),
        ),
        scratch_types=[
            pltpu.VMEM((E, TOK_PER_TILE), jnp.float32),
            pltpu.VMEM((TOK_PER_TILE,), jnp.int32),
            pltpu.VMEM((TOK_PER_TILE,), jnp.float32),
            pltpu.VMEM((N_TOKENS,), jnp.int32),
            pltpu.VMEM((PG,), jnp.int32),
            pltpu.VMEM((N_TOKENS,), jnp.int32),
            pltpu.VMEM((_LANES,), jnp.int32),
            pltpu.VMEM((N_GCH, GCH), jnp.int32),
            pltpu.VMEM((GCH, DMODEL), jnp.float32),
            pltpu.VMEM((GCH, DMODEL), jnp.float32),
            pltpu.VMEM_SHARED((N_TOKENS,), jnp.int32),
            pltpu.VMEM_SHARED((PG,), jnp.int32),
            pltpu.SemaphoreType.DMA,
            pltpu.SemaphoreType.DMA,
        ],
    )(_sc_dispatch_kernel)
    return f(lt3, x)


def _gffn_kernel(be_ref, xs_ref, w1_ref, w3_ref, w2_ref, ys_ref):
    xb = xs_ref[...].astype(jnp.bfloat16)
    a = jax.lax.dot_general(xb, w1_ref[0].astype(jnp.bfloat16),
                            (((1,), (1,)), ((), ())),
                            preferred_element_type=jnp.float32)
    b = jax.lax.dot_general(xb, w3_ref[0].astype(jnp.bfloat16),
                            (((1,), (1,)), ((), ())),
                            preferred_element_type=jnp.float32)
    h = (a * jax.nn.sigmoid(a) * b).astype(jnp.bfloat16)
    ys_ref[...] = jax.lax.dot_general(h, w2_ref[0].astype(jnp.bfloat16),
                                      (((1,), (1,)), ((), ())),
                                      preferred_element_type=jnp.float32)


def _sc_combine_kernel(ys_hbm, pos_hbm, w_hbm, out_hbm,
                       p_v, wv_v, r0_v, r1_v, o_v, sem, sem_st):
    wid = lax.axis_index("s") * _NC + lax.axis_index("c")
    tb = wid * TOK_D
    for gg in range(NGROUPS):
        pltpu.sync_copy(pos_hbm.at[gg, pl.ds(tb, TOK_D)], p_v.at[gg])
        pltpu.sync_copy(w_hbm.at[gg, pl.ds(tb, TOK_D)], wv_v.at[gg])
    for ch in range(TOK_D // CCH):
        pltpu.async_copy(ys_hbm.at[p_v.at[0, pl.ds(ch * CCH, CCH)]],
                         r0_v, sem).wait()
        pltpu.async_copy(ys_hbm.at[p_v.at[1, pl.ds(ch * CCH, CCH)]],
                         r1_v, sem).wait()
        wvec0 = [wv_v[0, pl.ds(ch * CCH + q * _LANES, _LANES)]
                 for q in range(CCH // _LANES)]
        wvec1 = [wv_v[1, pl.ds(ch * CCH + q * _LANES, _LANES)]
                 for q in range(CCH // _LANES)]
        for r in range(CCH):
            w0s = wvec0[r // _LANES][r % _LANES]
            w1s = wvec1[r // _LANES][r % _LANES]

            def _row(cc, c, r=r, w0s=w0s, w1s=w1s):
                for u in range(4):
                    sl = pl.ds((cc * 4 + u) * _LANES, _LANES)
                    o_v[r, sl] = w0s * r0_v[r, sl] + w1s * r1_v[r, sl]
                return c
            lax.fori_loop(0, DMODEL // (_LANES * 4), _row, 0)
        pltpu.sync_copy(o_v, out_hbm.at[pl.ds(tb + ch * CCH, CCH), :])


def _sc_combine(ys, pos, w):
    mesh = plsc.VectorSubcoreMesh(core_axis_name="c", subcore_axis_name="s")
    f = functools.partial(
        pl.kernel,
        mesh=mesh,
        compiler_params=pltpu.CompilerParams(needs_layout_passes=False),
        out_type=jax.ShapeDtypeStruct((N_TOKENS, DMODEL), jnp.float32),
        scratch_types=[
            pltpu.VMEM((NGROUPS, TOK_D), jnp.int32),
            pltpu.VMEM((NGROUPS, TOK_D), jnp.float32),
            pltpu.VMEM((CCH, DMODEL), jnp.float32),
            pltpu.VMEM((CCH, DMODEL), jnp.float32),
            pltpu.VMEM((CCH, DMODEL), jnp.float32),
            pltpu.SemaphoreType.DMA,
            pltpu.SemaphoreType.DMA,
        ],
    )(_sc_combine_kernel)
    return f(ys, pos, w)


@jax.jit
def kernel(hidden_states, gate_w, w1, w3, w2):
    lt3 = pl.pallas_call(
        _logits_kernel,
        out_shape=jax.ShapeDtypeStruct((_NS, E, TOK_PER_TILE), jnp.float32),
    )(hidden_states, gate_w)

    xs, pos, w, be = _sc_dispatch(lt3, hidden_states)

    ys = pl.pallas_call(
        _gffn_kernel,
        grid_spec=pltpu.PrefetchScalarGridSpec(
            num_scalar_prefetch=1,
            grid=(NB,),
            in_specs=[
                pl.BlockSpec((T_BLK, DMODEL), lambda i, be: (i, 0)),
                pl.BlockSpec((1, DFF, DMODEL),
                             lambda i, be: (be[i // NBG, i % NBG], 0, 0)),
                pl.BlockSpec((1, DFF, DMODEL),
                             lambda i, be: (be[i // NBG, i % NBG], 0, 0)),
                pl.BlockSpec((1, DMODEL, DFF),
                             lambda i, be: (be[i // NBG, i % NBG], 0, 0)),
            ],
            out_specs=pl.BlockSpec((T_BLK, DMODEL), lambda i, be: (i, 0)),
        ),
        out_shape=jax.ShapeDtypeStruct((PTOT, DMODEL), jnp.float32),
    )(be, xs, w1, w3, w2)

    return _sc_combine(ys, pos, w)


# speedup vs baseline: 2.1784x; 2.1620x over previous
import functools

import jax
import jax.numpy as jnp
from jax import lax
from jax.experimental import pallas as pl
from jax.experimental.pallas import tpu as pltpu
from jax.experimental.pallas import tpu_sc as plsc

E = 8
TOPK = 2
DMODEL = 1024
DFF = 512
NGROUPS = 2
GROUP_SIZE = E // NGROUPS
N_TOKENS = 2048

_SC_INFO = plsc.get_sparse_core_info()
_NC = _SC_INFO.num_cores
_NS = _SC_INFO.num_subcores
_NW = _NC * _NS
_LANES = _SC_INFO.num_lanes
_TOK_PER_W = N_TOKENS // _NW


def _logits_kernel(x_ref, gw_ref, lt_ref, xbf_ref):
    xbf_ref[...] = x_ref[...].astype(jnp.bfloat16)
    lt = jax.lax.dot_general(
        gw_ref[...], x_ref[...], (((0,), (1,)), ((), ())),
        preferred_element_type=jnp.float32)
    for wdx in range(_NW):
        lt_ref[wdx] = lt[:, wdx * _TOK_PER_W:(wdx + 1) * _TOK_PER_W]


def _sc_route_kernel(lt3_hbm, a0_hbm, a1_hbm, w0_hbm, w1_hbm,
                     lt_v, a0_v, a1_v, w0_v, w1_v):
    wid = lax.axis_index("s") * _NC + lax.axis_index("c")
    base = wid * _TOK_PER_W
    pltpu.sync_copy(lt3_hbm.at[wid], lt_v)
    for j in range(_TOK_PER_W // _LANES):
        sl = pl.ds(j * _LANES, _LANES)
        l = [lt_v[e, sl] for e in range(E)]
        m0 = l[0]
        a0 = jnp.full((_LANES,), 0, jnp.int32)
        m1 = l[GROUP_SIZE]
        a1 = jnp.full((_LANES,), GROUP_SIZE, jnp.int32)
        for i in range(1, GROUP_SIZE):
            gt0 = l[i] > m0
            a0 = jnp.where(gt0, i, a0)
            m0 = jnp.where(gt0, l[i], m0)
            gt1 = l[GROUP_SIZE + i] > m1
            a1 = jnp.where(gt1, GROUP_SIZE + i, a1)
            m1 = jnp.where(gt1, l[GROUP_SIZE + i], m1)
        mm = jnp.maximum(m0, m1)
        e0 = jnp.exp(m0 - mm)
        e1 = jnp.exp(m1 - mm)
        den = e0 + e1
        a0_v[sl] = a0
        a1_v[sl] = a1
        w0_v[sl] = e0 / den
        w1_v[sl] = e1 / den
    pltpu.sync_copy(a0_v, a0_hbm.at[pl.ds(base, _TOK_PER_W)])
    pltpu.sync_copy(a1_v, a1_hbm.at[pl.ds(base, _TOK_PER_W)])
    pltpu.sync_copy(w0_v, w0_hbm.at[pl.ds(base, _TOK_PER_W)])
    pltpu.sync_copy(w1_v, w1_hbm.at[pl.ds(base, _TOK_PER_W)])


def _sc_route(lt3):
    mesh = plsc.VectorSubcoreMesh(core_axis_name="c", subcore_axis_name="s")
    f = functools.partial(
        pl.kernel,
        mesh=mesh,
        compiler_params=pltpu.CompilerParams(needs_layout_passes=False),
        out_type=(
            jax.ShapeDtypeStruct((N_TOKENS,), jnp.int32),
            jax.ShapeDtypeStruct((N_TOKENS,), jnp.int32),
            jax.ShapeDtypeStruct((N_TOKENS,), jnp.float32),
            jax.ShapeDtypeStruct((N_TOKENS,), jnp.float32),
        ),
        scratch_types=[
            pltpu.VMEM((E, _TOK_PER_W), jnp.float32),
            pltpu.VMEM((_TOK_PER_W,), jnp.int32),
            pltpu.VMEM((_TOK_PER_W,), jnp.int32),
            pltpu.VMEM((_TOK_PER_W,), jnp.float32),
            pltpu.VMEM((_TOK_PER_W,), jnp.float32),
        ],
    )(_sc_route_kernel)
    return f(lt3)


def _ffn_kernel(a0_ref, a1_ref, w0_ref, w1_ref, x_ref, w1w_ref, w3w_ref,
                w2w_ref, out_ref):
    e = pl.program_id(0)
    comb = (jnp.where(a0_ref[...] == e, w0_ref[...], 0.0)
            + jnp.where(a1_ref[...] == e, w1_ref[...], 0.0))
    x = x_ref[...]
    a = jax.lax.dot_general(x, w1w_ref[0].astype(jnp.bfloat16),
                            (((1,), (1,)), ((), ())),
                            preferred_element_type=jnp.float32)
    b = jax.lax.dot_general(x, w3w_ref[0].astype(jnp.bfloat16),
                            (((1,), (1,)), ((), ())),
                            preferred_element_type=jnp.float32)
    h = (a * jax.nn.sigmoid(a) * b).astype(jnp.bfloat16)
    y = jax.lax.dot_general(h, w2w_ref[0].astype(jnp.bfloat16),
                            (((1,), (1,)), ((), ())),
                            preferred_element_type=jnp.float32)
    contrib = comb * y

    @pl.when(e == 0)
    def _():
        out_ref[...] = contrib

    @pl.when(e > 0)
    def _():
        out_ref[...] += contrib


@jax.jit
def kernel(hidden_states, gate_w, w1, w3, w2):
    lt3, x_bf = pl.pallas_call(
        _logits_kernel,
        out_shape=(
            jax.ShapeDtypeStruct((_NW, E, _TOK_PER_W), jnp.float32),
            jax.ShapeDtypeStruct((N_TOKENS, DMODEL), jnp.bfloat16),
        ),
    )(hidden_states, gate_w)

    a0, a1, wt0, wt1 = _sc_route(lt3)
    a0 = a0.reshape(N_TOKENS, 1)
    a1 = a1.reshape(N_TOKENS, 1)
    wt0 = wt0.reshape(N_TOKENS, 1)
    wt1 = wt1.reshape(N_TOKENS, 1)

    out = pl.pallas_call(
        _ffn_kernel,
        grid=(E,),
        in_specs=[
            pl.BlockSpec((N_TOKENS, 1), lambda e: (0, 0)),
            pl.BlockSpec((N_TOKENS, 1), lambda e: (0, 0)),
            pl.BlockSpec((N_TOKENS, 1), lambda e: (0, 0)),
            pl.BlockSpec((N_TOKENS, 1), lambda e: (0, 0)),
            pl.BlockSpec((N_TOKENS, DMODEL), lambda e: (0, 0)),
            pl.BlockSpec((1, DFF, DMODEL), lambda e: (e, 0, 0)),
            pl.BlockSpec((1, DFF, DMODEL), lambda e: (e, 0, 0)),
            pl.BlockSpec((1, DMODEL, DFF), lambda e: (e, 0, 0)),
        ],
        out_specs=pl.BlockSpec((N_TOKENS, DMODEL), lambda e: (0, 0)),
        out_shape=jax.ShapeDtypeStruct((N_TOKENS, DMODEL), jnp.float32),
    )(a0, a1, wt0, wt1, x_bf, w1, w3, w2)
    return out
